# Initial kernel scaffold; baseline (speedup 1.0000x reference)
import jax
import jax.numpy as jnp
from jax.experimental import pallas as pl


def kernel(features, labels):
    raise NotImplementedError("write your pallas kernel here")


# speedup vs baseline: 3.0747x; 3.0747x over previous
import jax
import jax.numpy as jnp
from jax.experimental import pallas as pl

K = 19


def _proto_kernel(feats_ref, labels_ref, out_ref, cnt_ref, *, bn):
    b = pl.program_id(0)
    j = pl.program_id(1)
    nb = pl.num_programs(1)
    nbt = pl.num_programs(0)

    feats = feats_ref[0]
    labels = labels_ref[0, 0]
    classes = jax.lax.broadcasted_iota(jnp.int32, (bn, K), 1)
    onehot = (labels[:, None] == classes).astype(jnp.float32)

    partial = jax.lax.dot_general(
        onehot, feats,
        dimension_numbers=(((0,), (1,)), ((), ())),
        preferred_element_type=jnp.float32,
    )
    cnt_partial = jnp.sum(onehot, axis=0).reshape(K, 1)

    @pl.when((b == 0) & (j == 0))
    def _init():
        out_ref[...] = partial
        cnt_ref[...] = cnt_partial

    @pl.when((b > 0) | (j > 0))
    def _acc():
        out_ref[...] += partial
        cnt_ref[...] += cnt_partial

    @pl.when((b == nbt - 1) & (j == nb - 1))
    def _finalize():
        cnt = cnt_ref[...]
        denom = jnp.maximum(cnt, 1.0)
        out_ref[...] = jnp.where(cnt > 0.0, out_ref[...] / denom,
                                 jnp.zeros_like(out_ref[...]))


def kernel(features, labels):
    B, C, H, W = features.shape
    N = H * W
    bn = 4096
    nb = N // bn

    feats3 = features.reshape(B, C, N)
    labels3 = labels.reshape(B * nb, 1, bn)

    sums, counts = pl.pallas_call(
        lambda f, l, o, c: _proto_kernel(f, l, o, c, bn=bn),
        grid=(B, nb),
        in_specs=[
            pl.BlockSpec((1, C, bn), lambda b, j: (b, 0, j)),
            pl.BlockSpec((1, 1, bn), lambda b, j: (b * nb + j, 0, 0)),
        ],
        out_specs=[
            pl.BlockSpec((K, C), lambda b, j: (0, 0)),
            pl.BlockSpec((K, 1), lambda b, j: (0, 0)),
        ],
        out_shape=[
            jax.ShapeDtypeStruct((K, C), jnp.float32),
            jax.ShapeDtypeStruct((K, 1), jnp.float32),
        ],
    )(feats3, labels3)

    return sums, counts.reshape(K)


# speedup vs baseline: 3.3066x; 1.0754x over previous
import jax
import jax.numpy as jnp
from jax.experimental import pallas as pl

K = 19


def _proto_kernel(feats_ref, labels_ref, out_ref, cnt_ref, *, bn):
    b = pl.program_id(0)
    j = pl.program_id(1)
    nb = pl.num_programs(1)
    nbt = pl.num_programs(0)

    feats = feats_ref[0]
    labels = labels_ref[0]
    classes = jax.lax.broadcasted_iota(jnp.int32, (K, bn), 0)
    onehot = (labels == classes).astype(jnp.float32)

    partial = jax.lax.dot_general(
        onehot, feats,
        dimension_numbers=(((1,), (1,)), ((), ())),
        preferred_element_type=jnp.float32,
    )
    cnt_partial = jnp.sum(onehot, axis=1).reshape(K, 1)

    @pl.when((b == 0) & (j == 0))
    def _init():
        out_ref[...] = partial
        cnt_ref[...] = cnt_partial

    @pl.when((b > 0) | (j > 0))
    def _acc():
        out_ref[...] += partial
        cnt_ref[...] += cnt_partial

    @pl.when((b == nbt - 1) & (j == nb - 1))
    def _finalize():
        cnt = cnt_ref[...]
        denom = jnp.maximum(cnt, 1.0)
        out_ref[...] = jnp.where(cnt > 0.0, out_ref[...] / denom,
                                 jnp.zeros_like(out_ref[...]))


def kernel(features, labels):
    B, C, H, W = features.shape
    N = H * W
    bn = 8192
    nb = N // bn

    feats3 = features.reshape(B, C, N)
    labels3 = labels.reshape(B * nb, 1, bn)

    sums, counts = pl.pallas_call(
        lambda f, l, o, c: _proto_kernel(f, l, o, c, bn=bn),
        grid=(B, nb),
        in_specs=[
            pl.BlockSpec((1, C, bn), lambda b, j: (b, 0, j)),
            pl.BlockSpec((1, 1, bn), lambda b, j: (b * nb + j, 0, 0)),
        ],
        out_specs=[
            pl.BlockSpec((K, C), lambda b, j: (0, 0)),
            pl.BlockSpec((K, 1), lambda b, j: (0, 0)),
        ],
        out_shape=[
            jax.ShapeDtypeStruct((K, C), jnp.float32),
            jax.ShapeDtypeStruct((K, 1), jnp.float32),
        ],
    )(feats3, labels3)

    return sums, counts.reshape(K)
